# Initial kernel scaffold; baseline (speedup 1.0000x reference)
import jax
import jax.numpy as jnp
from jax.experimental import pallas as pl


def kernel(user, feed, city, user_table, feed_table, city_table):
    raise NotImplementedError("write your pallas kernel here")


# speedup vs baseline: 1.0593x; 1.0593x over previous
import functools

import jax
import jax.numpy as jnp
from jax import lax
from jax.experimental import pallas as pl
from jax.experimental.pallas import tpu as pltpu
from jax.experimental.pallas import tpu_sc as plsc

B = 16384
D = 32
L = 16


def _sc_embed(user, feed, city, user_table, feed_table, city_table):
    info = plsc.get_sparse_core_info()
    nw = info.num_cores * info.num_subcores
    bpw = B // nw

    mesh = plsc.VectorSubcoreMesh(core_axis_name="c", subcore_axis_name="s")

    @functools.partial(
        pl.kernel,
        mesh=mesh,
        compiler_params=pltpu.CompilerParams(use_tc_tiling_on_sc=False),
        out_type=jax.ShapeDtypeStruct((3 * B, D), jnp.float32),
        scratch_types=[
            pltpu.VMEM((bpw,), jnp.int32),
            pltpu.VMEM((bpw,), jnp.int32),
            pltpu.VMEM((bpw,), jnp.int32),
            pltpu.VMEM((bpw,), jnp.int32),
            pltpu.VMEM((bpw,), jnp.int32),
            pltpu.VMEM((bpw,), jnp.int32),
            pltpu.VMEM((bpw, D), jnp.float32),
            pltpu.VMEM((bpw, D), jnp.float32),
            pltpu.VMEM((bpw, D), jnp.float32),
            pltpu.SemaphoreType.DMA,
            pltpu.SemaphoreType.DMA,
            pltpu.SemaphoreType.DMA,
            pltpu.SemaphoreType.DMA,
            pltpu.SemaphoreType.DMA,
            pltpu.SemaphoreType.DMA,
        ],
    )
    def k(user_hbm, feed_hbm, city_hbm, ut_hbm, ft_hbm, ct_hbm, out_hbm,
          uidx_v, fidx_v, cidx_v, udst_v, fdst_v, cdst_v,
          urows_v, frows_v, crows_v, su, sf, sc, pu, pf, pc):
        wid = lax.axis_index("s") * info.num_cores + lax.axis_index("c")
        base = wid * bpw
        pltpu.sync_copy(user_hbm.at[pl.ds(base, bpw)], uidx_v)
        pltpu.sync_copy(feed_hbm.at[pl.ds(base, bpw)], fidx_v)
        pltpu.sync_copy(city_hbm.at[pl.ds(base, bpw)], cidx_v)
        cu = pltpu.async_copy(ut_hbm.at[uidx_v], urows_v, su)
        cf = pltpu.async_copy(ft_hbm.at[fidx_v], frows_v, sf)
        cc = pltpu.async_copy(ct_hbm.at[cidx_v], crows_v, sc)

        tri_iota = lax.iota(jnp.int32, L) * 3
        for i in range(bpw // L):
            d = tri_iota + (3 * base + 3 * L * i)
            udst_v[pl.ds(L * i, L)] = d
            fdst_v[pl.ds(L * i, L)] = d + 1
            cdst_v[pl.ds(L * i, L)] = d + 2

        cu.wait()
        wu = pltpu.async_copy(urows_v, out_hbm.at[udst_v], pu)
        cf.wait()
        wf = pltpu.async_copy(frows_v, out_hbm.at[fdst_v], pf)
        cc.wait()
        wc = pltpu.async_copy(crows_v, out_hbm.at[cdst_v], pc)
        wu.wait()
        wf.wait()
        wc.wait()

    out3 = k(user, feed, city, user_table, feed_table, city_table)
    return out3.reshape(B, 3 * D)


def kernel(user, feed, city, user_table, feed_table, city_table):
    return _sc_embed(user.astype(jnp.int32), feed.astype(jnp.int32),
                     city.astype(jnp.int32), user_table, feed_table, city_table)
